# baseline (speedup 1.0000x reference)
import functools

import numpy as np
import jax
import jax.numpy as jnp
from jax import lax
from jax.experimental import pallas as pl
from jax.experimental.pallas import tpu as pltpu
from jax.experimental.pallas import tpu_sc as plsc

N = 10000
E = 320000
D_IN = 128
H1 = 128
H2 = 64
NC_OUT = 10

NCORES = 2
NSUB = 16
NWORK = NCORES * NSUB
CHUNK = 128
NCHUNKS = E // CHUNK
CPW = NCHUNKS // NWORK
NEXTRA = NCHUNKS - CPW * NWORK
IB = CPW // 3
AL_MAX = (NCHUNKS - (IB + 14)) // 8 * 8
CPWS = CPW + 18
ALD_MAX = (NCHUNKS - CPWS) // 8 * 8
NPAD = 10240
ROWS_PT = NPAD // NSUB
RB_CHUNKS = ROWS_PT // CHUNK

_mesh = plsc.VectorSubcoreMesh(core_axis_name="c", subcore_axis_name="s")

@functools.partial(
    pl.kernel,
    out_type=jax.ShapeDtypeStruct((NCORES, NPAD), jnp.float32),
    mesh=_mesh,
    scratch_types=[
        pltpu.VMEM_SHARED((NPAD,), jnp.float32),
        pltpu.VMEM((CPWS, CHUNK), jnp.int32),
        pltpu.VMEM((CHUNK,), jnp.float32),
        pltpu.VMEM((ROWS_PT,), jnp.float32),
    ],
)
def _deg_kernel(dst_hbm, dtail_hbm, deg_out, acc, dst_v, ones_v, buf_v):
    cc = lax.axis_index("c")
    ss = lax.axis_index("s")
    w = cc * NSUB + ss

    base = CPW * w
    al = jnp.minimum(base - lax.rem(base, 8), ALD_MAX)
    off = base - al
    pltpu.sync_copy(dst_hbm.at[pl.ds(pl.multiple_of(al, 8), CPWS)], dst_v)
    for i in range(ROWS_PT // 16):
        buf_v[pl.ds(i * 16, 16)] = jnp.zeros((16,), jnp.float32)
    for i in range(CHUNK // 16):
        ones_v[pl.ds(i * 16, 16)] = jnp.ones((16,), jnp.float32)
    pltpu.sync_copy(buf_v, acc.at[pl.ds(ROWS_PT * ss, ROWS_PT)])
    plsc.subcore_barrier()

    @pl.loop(0, CPW)
    def _(i):
        pltpu.sync_copy(ones_v, acc.at[dst_v.at[off + i]], add=True)

    @pl.when(ss < NEXTRA // NCORES)
    def _():
        ex = cc * (NEXTRA // NCORES) + ss
        pltpu.sync_copy(dtail_hbm, dst_v.at[pl.ds(0, 2 * NEXTRA)])
        pltpu.sync_copy(ones_v, acc.at[dst_v.at[ex]], add=True)

    plsc.subcore_barrier()
    pltpu.sync_copy(acc.at[pl.ds(ROWS_PT * ss, ROWS_PT)], buf_v)
    pltpu.sync_copy(buf_v, deg_out.at[cc, pl.ds(ROWS_PT * ss, ROWS_PT)])


def _make_agg_kernel(width):
    @functools.partial(
        pl.kernel,
        out_type=jax.ShapeDtypeStruct((NCORES, NPAD, width), jnp.float32),
        mesh=_mesh,
        scratch_types=[
            pltpu.VMEM_SHARED((NPAD, width), jnp.float32),
            pltpu.VMEM((IB + 14, CHUNK), jnp.int32),
            pltpu.VMEM((IB + 14, CHUNK), jnp.int32),
            pltpu.VMEM((CHUNK, width), jnp.float32),
            pltpu.VMEM((CHUNK, width), jnp.float32),
            pltpu.SemaphoreType.DMA,
            pltpu.SemaphoreType.DMA,
        ],
    )
    def _agg(g_hbm, src_hbm, dst_hbm, stail_hbm, dtail_hbm, zz_hbm, out, acc,
             src_v, dst_v, rows0, rows1, sem0, sem1):
        cc = lax.axis_index("c")
        ss = lax.axis_index("s")

        pltpu.sync_copy(zz_hbm, rows0)
        for k in range(RB_CHUNKS):
            pltpu.sync_copy(rows0, acc.at[pl.ds(ROWS_PT * ss + CHUNK * k, CHUNK)])
        plsc.subcore_barrier()

        def run(start_chunk, nblk):
            for blk in range(nblk):
                base = start_chunk + IB * blk
                al = jnp.minimum(base - lax.rem(base, 8), AL_MAX)
                off = base - al
                al = pl.multiple_of(al, 8)
                pltpu.sync_copy(src_hbm.at[pl.ds(al, IB + 14)], src_v)
                pltpu.sync_copy(dst_hbm.at[pl.ds(al, IB + 14)], dst_v)
                pltpu.async_copy(g_hbm.at[src_v.at[off]], rows0, sem0)
                pltpu.async_copy(g_hbm.at[src_v.at[off + 1]], rows1, sem1)

                @pl.loop(0, IB // 2)
                def _(gi):
                    c0 = off + 2 * gi
                    c1 = off + 2 * gi + 1
                    pltpu.make_async_copy(g_hbm.at[src_v.at[c0]], rows0, sem0).wait()
                    pltpu.sync_copy(rows0, acc.at[dst_v.at[c0]], add=True)

                    @pl.when(2 * gi + 2 < IB)
                    def _():
                        pltpu.async_copy(g_hbm.at[src_v.at[c0 + 2]], rows0, sem0)

                    pltpu.make_async_copy(g_hbm.at[src_v.at[c1]], rows1, sem1).wait()
                    pltpu.sync_copy(rows1, acc.at[dst_v.at[c1]], add=True)

                    @pl.when(2 * gi + 3 < IB)
                    def _():
                        pltpu.async_copy(g_hbm.at[src_v.at[c1 + 2]], rows1, sem1)

        w = cc * NSUB + ss
        run(w * CPW, CPW // IB)

        @pl.when(ss < NEXTRA // NCORES)
        def _():
            ex = cc * (NEXTRA // NCORES) + ss
            pltpu.sync_copy(stail_hbm, src_v.at[pl.ds(0, 2 * NEXTRA)])
            pltpu.sync_copy(dtail_hbm, dst_v.at[pl.ds(0, 2 * NEXTRA)])
            pltpu.async_copy(g_hbm.at[src_v.at[ex]], rows0, sem0)
            pltpu.make_async_copy(g_hbm.at[src_v.at[ex]], rows0, sem0).wait()
            pltpu.sync_copy(rows0, acc.at[dst_v.at[ex]], add=True)

        plsc.subcore_barrier()
        for k in range(RB_CHUNKS):
            base = ROWS_PT * ss + CHUNK * k
            pltpu.async_copy(acc.at[pl.ds(base, CHUNK)],
                             out.at[cc, pl.ds(base, CHUNK)], sem0)
        for k in range(RB_CHUNKS):
            base = ROWS_PT * ss + CHUNK * k
            pltpu.make_async_copy(acc.at[pl.ds(base, CHUNK)],
                                  out.at[cc, pl.ds(base, CHUNK)], sem0).wait()

    return _agg


_agg128 = _make_agg_kernel(H1)
H2P = 128


_BLK = 1000
_GRID = N // _BLK


def _tc1_body(p0, p1, x, w1, g1_out, dinv_out):
    deg = p0[...] + p1[...] + 1.0
    dinv = lax.rsqrt(deg)
    h = jnp.dot(x[...], w1[...], preferred_element_type=jnp.float32)
    g1_out[...] = h * dinv
    dinv_out[...] = dinv


def _tc2_body(p, g1, dinv, b1, w2, g2_out):
    pv = p[...]
    s = (pv[0] + pv[1] + g1[...]) * dinv[...]
    h1 = jnp.maximum(s + b1[...], 0.0)
    h2 = jnp.dot(h1, w2[...], preferred_element_type=jnp.float32)
    g2_out[...] = h2 * dinv[...]


def _tc3_body(q, g2, dinv, b2, wc, bc, out):
    qv = q[...]
    s = (qv[0] + qv[1] + g2[...]) * dinv[...]
    h2 = jnp.maximum(s + b2[...], 0.0)
    out[...] = jnp.dot(h2, wc[...], preferred_element_type=jnp.float32) + bc[...]


def _row_spec(width):
    return pl.BlockSpec((_BLK, width), lambda i: (i, 0))


def _pair_spec(width):
    return pl.BlockSpec((NCORES, _BLK, width), lambda i: (0, i, 0))


def _full_spec(a, b):
    return pl.BlockSpec((a, b), lambda i: (0, 0))


_tc1 = pl.pallas_call(
    _tc1_body,
    grid=(_GRID,),
    in_specs=[_row_spec(1), _row_spec(1), _row_spec(D_IN), _full_spec(D_IN, H1)],
    out_specs=[_row_spec(H1), _row_spec(1)],
    out_shape=[
        jax.ShapeDtypeStruct((N, H1), jnp.float32),
        jax.ShapeDtypeStruct((N, 1), jnp.float32),
    ],
)

_tc2 = pl.pallas_call(
    _tc2_body,
    grid=(_GRID,),
    in_specs=[_pair_spec(H1), _row_spec(H1), _row_spec(1),
              _full_spec(1, H1), _full_spec(H1, H2P)],
    out_specs=[_row_spec(H2P)],
    out_shape=[jax.ShapeDtypeStruct((N, H2P), jnp.float32)],
)

_tc3 = pl.pallas_call(
    _tc3_body,
    grid=(_GRID,),
    in_specs=[_pair_spec(H2P), _row_spec(H2P), _row_spec(1),
              _full_spec(1, H2P), _full_spec(H2P, NC_OUT), _full_spec(1, NC_OUT)],
    out_specs=[_row_spec(NC_OUT)],
    out_shape=[jax.ShapeDtypeStruct((N, NC_OUT), jnp.float32)],
)


def kernel(x, edge_index, W1, b1, W2, b2, Wc, bc):
    src_r = edge_index[0].reshape(NCHUNKS, CHUNK)
    dst_r = edge_index[1].reshape(NCHUNKS, CHUNK)
    s_tail = jnp.concatenate([src_r[NWORK * CPW:], src_r[NWORK * CPW:]])
    d_tail = jnp.concatenate([dst_r[NWORK * CPW:], dst_r[NWORK * CPW:]])
    zz = jnp.zeros((CHUNK, H1), jnp.float32)

    degp = _deg_kernel(dst_r, d_tail)
    p0 = degp[0, :N].reshape(N, 1)
    p1 = degp[1, :N].reshape(N, 1)

    W2p = jnp.pad(W2, ((0, 0), (0, H2P - H2)))
    b2p = jnp.pad(b2, (0, H2P - H2)).reshape(1, H2P)
    Wcp = jnp.pad(Wc, ((0, H2P - H2), (0, 0)))

    g1, dinv = _tc1(p0, p1, x, W1)
    part1 = _agg128(g1, src_r, dst_r, s_tail, d_tail, zz)
    (g2,) = _tc2(part1, g1, dinv, b1.reshape(1, H1), W2p)
    part2 = _agg128(g2, src_r, dst_r, s_tail, d_tail, zz)
    (out,) = _tc3(part2, g2, dinv, b2p, Wcp, bc.reshape(1, NC_OUT))
    return out


# speedup vs baseline: 1.0770x; 1.0770x over previous
import functools

import numpy as np
import jax
import jax.numpy as jnp
from jax import lax
from jax.experimental import pallas as pl
from jax.experimental.pallas import tpu as pltpu
from jax.experimental.pallas import tpu_sc as plsc

N = 10000
E = 320000
D_IN = 128
H1 = 128
H2 = 64
NC_OUT = 10

NCORES = 2
NSUB = 16
NWORK = NCORES * NSUB
CHUNK = 128
NCHUNKS = E // CHUNK
CPW = NCHUNKS // NWORK
NEXTRA = NCHUNKS - CPW * NWORK
EPW = CPW * CHUNK
IB = CPW // 3
CPWP = 80
NPAD = 10240
ROWS_PT = NPAD // NSUB
RB_CHUNKS = ROWS_PT // CHUNK

_mesh = plsc.VectorSubcoreMesh(core_axis_name="c", subcore_axis_name="s")

@functools.partial(
    pl.kernel,
    out_type=[
        jax.ShapeDtypeStruct((NCORES, NPAD), jnp.float32),
        jax.ShapeDtypeStruct((NWORK, CPWP, CHUNK), jnp.int32),
    ],
    mesh=_mesh,
    scratch_types=[
        pltpu.VMEM_SHARED((NPAD,), jnp.float32),
        pltpu.VMEM((EPW + CHUNK,), jnp.int32),
        pltpu.VMEM((CPWP, CHUNK), jnp.int32),
        pltpu.VMEM((CHUNK,), jnp.float32),
        pltpu.VMEM((ROWS_PT,), jnp.float32),
    ],
)
def _deg_kernel(edge_hbm, deg_out, dl_out, acc, d1_v, dst_v, ones_v, buf_v):
    cc = lax.axis_index("c")
    ss = lax.axis_index("s")
    w = cc * NSUB + ss

    pltpu.sync_copy(edge_hbm.at[1, pl.ds(EPW * w, EPW)], d1_v.at[pl.ds(0, EPW)])

    @pl.when(w < NEXTRA)
    def _():
        pltpu.sync_copy(edge_hbm.at[1, pl.ds(NWORK * EPW + CHUNK * w, CHUNK)],
                        d1_v.at[pl.ds(EPW, CHUNK)])

    @pl.loop(0, CPW)
    def _(r):
        for i in range(CHUNK // 16):
            dst_v[r, pl.ds(16 * i, 16)] = d1_v[pl.ds(CHUNK * r + 16 * i, 16)]

    @pl.when(w < NEXTRA)
    def _():
        for i in range(CHUNK // 16):
            dst_v[CPW, pl.ds(16 * i, 16)] = d1_v[pl.ds(EPW + 16 * i, 16)]

    for i in range(ROWS_PT // 16):
        buf_v[pl.ds(i * 16, 16)] = jnp.zeros((16,), jnp.float32)
    for i in range(CHUNK // 16):
        ones_v[pl.ds(i * 16, 16)] = jnp.ones((16,), jnp.float32)
    pltpu.sync_copy(buf_v, acc.at[pl.ds(ROWS_PT * ss, ROWS_PT)])
    plsc.subcore_barrier()

    @pl.loop(0, CPW)
    def _(i):
        pltpu.sync_copy(ones_v, acc.at[dst_v.at[i]], add=True)

    @pl.when(w < NEXTRA)
    def _():
        pltpu.sync_copy(ones_v, acc.at[dst_v.at[CPW]], add=True)

    plsc.subcore_barrier()
    pltpu.sync_copy(acc.at[pl.ds(ROWS_PT * ss, ROWS_PT)], buf_v)
    pltpu.sync_copy(buf_v, deg_out.at[cc, pl.ds(ROWS_PT * ss, ROWS_PT)])
    pltpu.sync_copy(dst_v, dl_out.at[w])


def _make_agg_kernel(width):
    @functools.partial(
        pl.kernel,
        out_type=jax.ShapeDtypeStruct((NCORES, NPAD, width), jnp.float32),
        mesh=_mesh,
        scratch_types=[
            pltpu.VMEM_SHARED((NPAD, width), jnp.float32),
            pltpu.VMEM((IB * CHUNK,), jnp.int32),
            pltpu.VMEM((CPWP, CHUNK), jnp.int32),
            pltpu.VMEM((CHUNK, width), jnp.float32),
            pltpu.VMEM((CHUNK, width), jnp.float32),
            pltpu.SemaphoreType.DMA,
            pltpu.SemaphoreType.DMA,
        ],
    )
    def _agg(g_hbm, edge_hbm, dl_hbm, out, acc,
             src_v, dst_v, rows0, rows1, sem0, sem1):
        cc = lax.axis_index("c")
        ss = lax.axis_index("s")
        w = cc * NSUB + ss

        @pl.loop(0, CHUNK)
        def _(r):
            for i in range(width // 16):
                rows0[r, pl.ds(16 * i, 16)] = jnp.zeros((16,), jnp.float32)

        for k in range(RB_CHUNKS):
            pltpu.sync_copy(rows0, acc.at[pl.ds(ROWS_PT * ss + CHUNK * k, CHUNK)])
        pltpu.sync_copy(dl_hbm.at[w], dst_v)
        plsc.subcore_barrier()

        for blk in range(CPW // IB):
            pltpu.sync_copy(edge_hbm.at[0, pl.ds(EPW * w + IB * CHUNK * blk,
                                                 IB * CHUNK)], src_v)
            sref = lambda j: src_v.at[pl.ds(CHUNK * j, CHUNK)]
            pltpu.async_copy(g_hbm.at[sref(0)], rows0, sem0)
            pltpu.async_copy(g_hbm.at[sref(1)], rows1, sem1)

            @pl.loop(0, IB // 2)
            def _(gi):
                j0 = 2 * gi
                j1 = 2 * gi + 1
                pltpu.make_async_copy(g_hbm.at[sref(j0)], rows0, sem0).wait()
                pltpu.sync_copy(rows0, acc.at[dst_v.at[IB * blk + j0]], add=True)

                @pl.when(j0 + 2 < IB)
                def _():
                    pltpu.async_copy(g_hbm.at[sref(j0 + 2)], rows0, sem0)

                pltpu.make_async_copy(g_hbm.at[sref(j1)], rows1, sem1).wait()
                pltpu.sync_copy(rows1, acc.at[dst_v.at[IB * blk + j1]], add=True)

                @pl.when(j1 + 2 < IB)
                def _():
                    pltpu.async_copy(g_hbm.at[sref(j1 + 2)], rows1, sem1)

        @pl.when(w < NEXTRA)
        def _():
            pltpu.sync_copy(edge_hbm.at[0, pl.ds(NWORK * EPW + CHUNK * w, CHUNK)],
                            src_v.at[pl.ds(0, CHUNK)])
            pltpu.async_copy(g_hbm.at[src_v.at[pl.ds(0, CHUNK)]], rows0, sem0)
            pltpu.make_async_copy(g_hbm.at[src_v.at[pl.ds(0, CHUNK)]],
                                  rows0, sem0).wait()
            pltpu.sync_copy(rows0, acc.at[dst_v.at[CPW]], add=True)

        plsc.subcore_barrier()
        for k in range(RB_CHUNKS):
            base = ROWS_PT * ss + CHUNK * k
            pltpu.async_copy(acc.at[pl.ds(base, CHUNK)],
                             out.at[cc, pl.ds(base, CHUNK)], sem0)
        for k in range(RB_CHUNKS):
            base = ROWS_PT * ss + CHUNK * k
            pltpu.make_async_copy(acc.at[pl.ds(base, CHUNK)],
                                  out.at[cc, pl.ds(base, CHUNK)], sem0).wait()

    return _agg


_agg128 = _make_agg_kernel(H1)
H2P = 128


_BLK = 1000
_GRID = N // _BLK


def _tc0_body(x, w1, h1_out):
    h1_out[...] = jnp.dot(x[...], w1[...], preferred_element_type=jnp.float32)


def _tc1_body(p0, p1, h, g1_out, dinv_out):
    deg = p0[...] + p1[...] + 1.0
    dinv = lax.rsqrt(deg)
    g1_out[...] = h[...] * dinv
    dinv_out[...] = dinv


def _tc2_body(p, g1, dinv, b1, w2, g2_out):
    pv = p[...]
    s = (pv[0] + pv[1] + g1[...]) * dinv[...]
    h1 = jnp.maximum(s + b1[...], 0.0)
    h2 = jnp.dot(h1, w2[...], preferred_element_type=jnp.float32)
    g2_out[...] = h2 * dinv[...]


def _tc3_body(q, g2, dinv, b2, wc, bc, out):
    qv = q[...]
    s = (qv[0] + qv[1] + g2[...]) * dinv[...]
    h2 = jnp.maximum(s + b2[...], 0.0)
    out[...] = jnp.dot(h2, wc[...], preferred_element_type=jnp.float32) + bc[...]


def _row_spec(width):
    return pl.BlockSpec((_BLK, width), lambda i: (i, 0))


def _pair_spec(width):
    return pl.BlockSpec((NCORES, _BLK, width), lambda i: (0, i, 0))


def _full_spec(a, b):
    return pl.BlockSpec((a, b), lambda i: (0, 0))


_tc0 = pl.pallas_call(
    _tc0_body,
    grid=(_GRID,),
    in_specs=[_row_spec(D_IN), _full_spec(D_IN, H1)],
    out_specs=[_row_spec(H1)],
    out_shape=[jax.ShapeDtypeStruct((N, H1), jnp.float32)],
)

_tc1 = pl.pallas_call(
    _tc1_body,
    grid=(_GRID,),
    in_specs=[_row_spec(1), _row_spec(1), _row_spec(H1)],
    out_specs=[_row_spec(H1), _row_spec(1)],
    out_shape=[
        jax.ShapeDtypeStruct((N, H1), jnp.float32),
        jax.ShapeDtypeStruct((N, 1), jnp.float32),
    ],
)

_tc2 = pl.pallas_call(
    _tc2_body,
    grid=(_GRID,),
    in_specs=[_pair_spec(H1), _row_spec(H1), _row_spec(1),
              _full_spec(1, H1), _full_spec(H1, H2P)],
    out_specs=[_row_spec(H2P)],
    out_shape=[jax.ShapeDtypeStruct((N, H2P), jnp.float32)],
)

_tc3 = pl.pallas_call(
    _tc3_body,
    grid=(_GRID,),
    in_specs=[_pair_spec(H2P), _row_spec(H2P), _row_spec(1),
              _full_spec(1, H2P), _full_spec(H2P, NC_OUT), _full_spec(1, NC_OUT)],
    out_specs=[_row_spec(NC_OUT)],
    out_shape=[jax.ShapeDtypeStruct((N, NC_OUT), jnp.float32)],
)


def kernel(x, edge_index, W1, b1, W2, b2, Wc, bc):
    degp, dst_lin = _deg_kernel(edge_index)
    p0 = degp[0, :N].reshape(N, 1)
    p1 = degp[1, :N].reshape(N, 1)

    W2p = jnp.pad(W2, ((0, 0), (0, H2P - H2)))
    b2p = jnp.pad(b2, (0, H2P - H2)).reshape(1, H2P)
    Wcp = jnp.pad(Wc, ((0, H2P - H2), (0, 0)))

    (h1,) = _tc0(x, W1)
    g1, dinv = _tc1(p0, p1, h1)
    part1 = _agg128(g1, edge_index, dst_lin)
    (g2,) = _tc2(part1, g1, dinv, b1.reshape(1, H1), W2p)
    part2 = _agg128(g2, edge_index, dst_lin)
    (out,) = _tc3(part2, g2, dinv, b2p, Wcp, bc.reshape(1, NC_OUT))
    return out


# speedup vs baseline: 1.0806x; 1.0033x over previous
import functools

import jax
import jax.numpy as jnp
from jax import lax
from jax.experimental import pallas as pl
from jax.experimental.pallas import tpu as pltpu
from jax.experimental.pallas import tpu_sc as plsc

N = 10000
E = 320000
D_IN = 128
H1 = 128
H2 = 64
NC_OUT = 10

NCORES = 2
NSUB = 16
NWORK = NCORES * NSUB
CHUNK = 128
NCHUNKS = E // CHUNK
CPW = NCHUNKS // NWORK
NEXTRA = NCHUNKS - CPW * NWORK
EPW = CPW * CHUNK
IB = CPW // 3
CPWP = 80
NPAD = 10240
ROWS_PT = NPAD // NSUB
RB_CHUNKS = ROWS_PT // CHUNK

_mesh = plsc.VectorSubcoreMesh(core_axis_name="c", subcore_axis_name="s")

@functools.partial(
    pl.kernel,
    out_type=[
        jax.ShapeDtypeStruct((NCORES, NPAD), jnp.float32),
        jax.ShapeDtypeStruct((NWORK, CPWP, CHUNK), jnp.int32),
    ],
    mesh=_mesh,
    scratch_types=[
        pltpu.VMEM_SHARED((NPAD,), jnp.float32),
        pltpu.VMEM((EPW + CHUNK,), jnp.int32),
        pltpu.VMEM((CPWP, CHUNK), jnp.int32),
        pltpu.VMEM((CHUNK,), jnp.float32),
        pltpu.VMEM((ROWS_PT,), jnp.float32),
    ],
)
def _deg_kernel(edge_hbm, deg_out, dl_out, acc, d1_v, dst_v, ones_v, buf_v):
    cc = lax.axis_index("c")
    ss = lax.axis_index("s")
    w = cc * NSUB + ss

    pltpu.sync_copy(edge_hbm.at[1, pl.ds(EPW * w, EPW)], d1_v.at[pl.ds(0, EPW)])

    @pl.when(w < NEXTRA)
    def _():
        pltpu.sync_copy(edge_hbm.at[1, pl.ds(NWORK * EPW + CHUNK * w, CHUNK)],
                        d1_v.at[pl.ds(EPW, CHUNK)])

    @pl.loop(0, CPW)
    def _(r):
        for i in range(CHUNK // 16):
            dst_v[r, pl.ds(16 * i, 16)] = d1_v[pl.ds(CHUNK * r + 16 * i, 16)]

    @pl.when(w < NEXTRA)
    def _():
        for i in range(CHUNK // 16):
            dst_v[CPW, pl.ds(16 * i, 16)] = d1_v[pl.ds(EPW + 16 * i, 16)]

    for i in range(ROWS_PT // 16):
        buf_v[pl.ds(i * 16, 16)] = jnp.zeros((16,), jnp.float32)
    for i in range(CHUNK // 16):
        ones_v[pl.ds(i * 16, 16)] = jnp.ones((16,), jnp.float32)
    pltpu.sync_copy(buf_v, acc.at[pl.ds(ROWS_PT * ss, ROWS_PT)])
    plsc.subcore_barrier()

    @pl.loop(0, CPW)
    def _(i):
        pltpu.sync_copy(ones_v, acc.at[dst_v.at[i]], add=True)

    @pl.when(w < NEXTRA)
    def _():
        pltpu.sync_copy(ones_v, acc.at[dst_v.at[CPW]], add=True)

    plsc.subcore_barrier()
    pltpu.sync_copy(acc.at[pl.ds(ROWS_PT * ss, ROWS_PT)], buf_v)
    pltpu.sync_copy(buf_v, deg_out.at[cc, pl.ds(ROWS_PT * ss, ROWS_PT)])
    pltpu.sync_copy(dst_v, dl_out.at[w])


def _make_agg_kernel(width):
    @functools.partial(
        pl.kernel,
        out_type=jax.ShapeDtypeStruct((NCORES, NPAD, width), jnp.float32),
        mesh=_mesh,
        scratch_types=[
            pltpu.VMEM_SHARED((NPAD, width), jnp.float32),
            pltpu.VMEM((IB * CHUNK,), jnp.int32),
            pltpu.VMEM((CPWP, CHUNK), jnp.int32),
            pltpu.VMEM((CHUNK, width), jnp.float32),
            pltpu.VMEM((CHUNK, width), jnp.float32),
            pltpu.SemaphoreType.DMA,
            pltpu.SemaphoreType.DMA,
        ],
    )
    def _agg(g_hbm, edge_hbm, dl_hbm, out, acc,
             src_v, dst_v, rows0, rows1, sem0, sem1):
        cc = lax.axis_index("c")
        ss = lax.axis_index("s")
        w = cc * NSUB + ss

        @pl.loop(0, CHUNK)
        def _(r):
            for i in range(width // 16):
                rows0[r, pl.ds(16 * i, 16)] = jnp.zeros((16,), jnp.float32)

        for k in range(RB_CHUNKS):
            pltpu.sync_copy(rows0, acc.at[pl.ds(ROWS_PT * ss + CHUNK * k, CHUNK)])
        pltpu.sync_copy(dl_hbm.at[w], dst_v)
        plsc.subcore_barrier()

        for blk in range(CPW // IB):
            pltpu.sync_copy(edge_hbm.at[0, pl.ds(EPW * w + IB * CHUNK * blk,
                                                 IB * CHUNK)], src_v)
            sref = lambda j: src_v.at[pl.ds(CHUNK * j, CHUNK)]
            pltpu.async_copy(g_hbm.at[sref(0)], rows0, sem0)
            pltpu.async_copy(g_hbm.at[sref(1)], rows1, sem1)

            @pl.loop(0, IB // 2)
            def _(gi):
                j0 = 2 * gi
                j1 = 2 * gi + 1
                pltpu.make_async_copy(g_hbm.at[sref(j0)], rows0, sem0).wait()
                pltpu.sync_copy(rows0, acc.at[dst_v.at[IB * blk + j0]], add=True)

                @pl.when(j0 + 2 < IB)
                def _():
                    pltpu.async_copy(g_hbm.at[sref(j0 + 2)], rows0, sem0)

                pltpu.make_async_copy(g_hbm.at[sref(j1)], rows1, sem1).wait()
                pltpu.sync_copy(rows1, acc.at[dst_v.at[IB * blk + j1]], add=True)

                @pl.when(j1 + 2 < IB)
                def _():
                    pltpu.async_copy(g_hbm.at[sref(j1 + 2)], rows1, sem1)

        @pl.when(w < NEXTRA)
        def _():
            pltpu.sync_copy(edge_hbm.at[0, pl.ds(NWORK * EPW + CHUNK * w, CHUNK)],
                            src_v.at[pl.ds(0, CHUNK)])
            pltpu.async_copy(g_hbm.at[src_v.at[pl.ds(0, CHUNK)]], rows0, sem0)
            pltpu.make_async_copy(g_hbm.at[src_v.at[pl.ds(0, CHUNK)]],
                                  rows0, sem0).wait()
            pltpu.sync_copy(rows0, acc.at[dst_v.at[CPW]], add=True)

        plsc.subcore_barrier()
        for k in range(RB_CHUNKS):
            base = ROWS_PT * ss + CHUNK * k
            pltpu.async_copy(acc.at[pl.ds(base, CHUNK)],
                             out.at[cc, pl.ds(base, CHUNK)], sem0)
        for k in range(RB_CHUNKS):
            base = ROWS_PT * ss + CHUNK * k
            pltpu.make_async_copy(acc.at[pl.ds(base, CHUNK)],
                                  out.at[cc, pl.ds(base, CHUNK)], sem0).wait()

    return _agg


_agg128 = _make_agg_kernel(H1)
H2P = 128


_BLK = 1000
_GRID = N // _BLK


def _tc0_body(x, w1, h1_out):
    h1_out[...] = jnp.dot(x[...], w1[...], preferred_element_type=jnp.float32)


def _tc1_body(p0, p1, h, g1_out, dinv_out):
    deg = p0[...] + p1[...] + 1.0
    dinv = lax.rsqrt(deg)
    g1_out[...] = h[...] * dinv
    dinv_out[...] = dinv


def _tc2_body(p, g1, dinv, b1, w2, g2_out):
    pv = p[...]
    s = (pv[0] + pv[1] + g1[...]) * dinv[...]
    h1 = jnp.maximum(s + b1[...], 0.0)
    h2 = jnp.dot(h1, w2[...], preferred_element_type=jnp.float32)
    g2_out[...] = h2 * dinv[...]


def _tc3_body(q, g2, dinv, b2, wc, bc, out):
    qv = q[...]
    s = (qv[0] + qv[1] + g2[...]) * dinv[...]
    h2 = jnp.maximum(s + b2[...], 0.0)
    out[...] = jnp.dot(h2, wc[...], preferred_element_type=jnp.float32) + bc[...]


def _row_spec(width):
    return pl.BlockSpec((_BLK, width), lambda i: (i, 0))


def _pair_spec(width):
    return pl.BlockSpec((NCORES, _BLK, width), lambda i: (0, i, 0))


def _full_spec(a, b):
    return pl.BlockSpec((a, b), lambda i: (0, 0))


_tc0 = pl.pallas_call(
    _tc0_body,
    grid=(_GRID,),
    in_specs=[_row_spec(D_IN), _full_spec(D_IN, H1)],
    out_specs=[_row_spec(H1)],
    out_shape=[jax.ShapeDtypeStruct((N, H1), jnp.float32)],
)

_tc1 = pl.pallas_call(
    _tc1_body,
    grid=(_GRID,),
    in_specs=[_row_spec(1), _row_spec(1), _row_spec(H1)],
    out_specs=[_row_spec(H1), _row_spec(1)],
    out_shape=[
        jax.ShapeDtypeStruct((N, H1), jnp.float32),
        jax.ShapeDtypeStruct((N, 1), jnp.float32),
    ],
)

_tc2 = pl.pallas_call(
    _tc2_body,
    grid=(_GRID,),
    in_specs=[_pair_spec(H1), _row_spec(H1), _row_spec(1),
              _full_spec(1, H1), _full_spec(H1, H2P)],
    out_specs=[_row_spec(H2P)],
    out_shape=[jax.ShapeDtypeStruct((N, H2P), jnp.float32)],
)

_tc3 = pl.pallas_call(
    _tc3_body,
    grid=(_GRID,),
    in_specs=[_pair_spec(H2P), _row_spec(H2P), _row_spec(1),
              _full_spec(1, H2P), _full_spec(H2P, NC_OUT), _full_spec(1, NC_OUT)],
    out_specs=[_row_spec(NC_OUT)],
    out_shape=[jax.ShapeDtypeStruct((N, NC_OUT), jnp.float32)],
)


def kernel(x, edge_index, W1, b1, W2, b2, Wc, bc):
    degp, dst_lin = _deg_kernel(edge_index)
    p0 = degp[0, :N].reshape(N, 1)
    p1 = degp[1, :N].reshape(N, 1)

    W2p = jnp.pad(W2, ((0, 0), (0, H2P - H2)))
    b2p = jnp.pad(b2, (0, H2P - H2)).reshape(1, H2P)
    Wcp = jnp.pad(Wc, ((0, H2P - H2), (0, 0)))

    (h1,) = _tc0(x, W1)
    g1, dinv = _tc1(p0, p1, h1)
    part1 = _agg128(g1, edge_index, dst_lin)
    (g2,) = _tc2(part1, g1, dinv, b1.reshape(1, H1), W2p)
    part2 = _agg128(g2, edge_index, dst_lin)
    (out,) = _tc3(part2, g2, dinv, b2p, Wcp, bc.reshape(1, NC_OUT))
    return out
